# baseline (speedup 1.0000x reference)
import functools

import jax
import jax.numpy as jnp
from jax import lax
from jax.experimental import pallas as pl
from jax.experimental.pallas import tpu as pltpu
from jax.experimental.pallas import tpu_sc as plsc

N = 10000
D = 128
E = 320000

NC = 2
NS = 16
NW = NC * NS
CHUNK = 128
EDGES_PER_TILE = E // NW
STEPS = 80
E_TILE_PAD = STEPS * CHUNK
PAD_SLOTS = E_TILE_PAD - EDGES_PER_TILE
NBUF = 2
GROUPS = STEPS // NBUF

ROW_CHUNK = 640
ROW_LAST = N - (NS - 1) * ROW_CHUNK
N_PAD = NS * ROW_CHUNK


def _sc_agg_body(with_deg, h_hbm, src_hbm, dst_hbm, z2d_hbm, *rest):
    if with_deg:
        (agg_out, deg_out, acc_sh, deg_sh,
         src0, src1, src2, src3, dst0, dst1, dst2, dst3,
         rows0, rows1, ones_v, zbuf_v,
         isem0, isem1, isem2, isem3, gsem0, gsem1, ssem0, ssem1) = rest
    else:
        (agg_out, acc_sh,
         src0, src1, src2, src3, dst0, dst1, dst2, dst3,
         rows0, rows1,
         isem0, isem1, isem2, isem3, gsem0, gsem1, ssem0, ssem1) = rest
    srcb = [src0, src1, src2, src3]
    dstb = [dst0, dst1, dst2, dst3]
    rows = [rows0, rows1]
    isems = [isem0, isem1, isem2, isem3]
    gsems = [gsem0, gsem1]
    ssems = [ssem0, ssem1]
    c = lax.axis_index("c")
    s = lax.axis_index("s")
    w = c * NS + s

    def rows_split(fn):
        @pl.when(s < NS - 1)
        def _():
            fn(s * ROW_CHUNK, ROW_CHUNK)

        @pl.when(s == NS - 1)
        def _():
            fn((NS - 1) * ROW_CHUNK, ROW_LAST)

    def idx_start(t, ij):
        pltpu.async_copy(src_hbm.at[w, t], srcb[ij], isems[ij])
        pltpu.async_copy(dst_hbm.at[w, t], dstb[ij], isems[ij])

    def idx_wait(t, ij):
        pltpu.make_async_copy(src_hbm.at[w, t], srcb[ij], isems[ij]).wait()
        pltpu.make_async_copy(dst_hbm.at[w, t], dstb[ij], isems[ij]).wait()

    def gather_start(ij, rj):
        pltpu.async_copy(h_hbm.at[srcb[ij]], rows[rj], gsems[rj])

    def gather_wait(ij, rj):
        pltpu.make_async_copy(h_hbm.at[srcb[ij]], rows[rj], gsems[rj]).wait()

    def scatter_start(ij, rj):
        pltpu.async_copy(rows[rj], acc_sh.at[dstb[ij]], ssems[rj], add=True)
        if with_deg:
            pltpu.async_copy(ones_v, deg_sh.at[dstb[ij]], ssems[rj],
                             add=True)

    def scatter_wait(ij, rj):
        pltpu.make_async_copy(rows[rj], acc_sh.at[dstb[ij]],
                              ssems[rj]).wait()
        if with_deg:
            pltpu.make_async_copy(ones_v, deg_sh.at[dstb[ij]],
                                  ssems[rj]).wait()

    pltpu.sync_copy(z2d_hbm.at[pl.ds(s * ROW_CHUNK, ROW_CHUNK), :],
                    acc_sh.at[pl.ds(s * ROW_CHUNK, ROW_CHUNK), :])
    if with_deg:
        for k in range(ROW_CHUNK // 16):
            zbuf_v[pl.ds(k * 16, 16)] = jnp.zeros((16,), jnp.float32)
        pltpu.sync_copy(zbuf_v, deg_sh.at[pl.ds(s * ROW_CHUNK, ROW_CHUNK)])

        for k in range(CHUNK // 16):
            ones_v[pl.ds(k * 16, 16)] = jnp.ones((16,), jnp.float32)

    plsc.subcore_barrier()

    idx_start(0, 0)
    idx_start(1, 1)

    def group(g, carry):
        for j in range(4):
            t = g * 4 + j
            idx_wait(t, j)

            @pl.when(t >= 2)
            def _():
                scatter_wait((j - 2) % 4, j % 2)

            gather_start(j, j % 2)

            @pl.when(t >= 1)
            def _():
                gather_wait((j - 1) % 4, (j - 1) % 2)
                scatter_start((j - 1) % 4, (j - 1) % 2)

            @pl.when(t + 2 < STEPS)
            def _():
                idx_start(t + 2, (j + 2) % 4)
        return carry

    lax.fori_loop(0, STEPS // 4, group, 0)
    gather_wait(3, 1)
    scatter_start(3, 1)
    scatter_wait(2, 0)
    scatter_wait(3, 1)

    plsc.subcore_barrier()

    rows_split(lambda b, n: pltpu.sync_copy(
        acc_sh.at[pl.ds(b, n), :], agg_out.at[c, pl.ds(b, n), :]))
    if with_deg:
        b = s * ROW_CHUNK
        pltpu.sync_copy(deg_sh.at[pl.ds(b, ROW_CHUNK)], zbuf_v)
        pltpu.sync_copy(zbuf_v, deg_out.at[c, pl.ds(b, ROW_CHUNK)])


def _sc_aggregate(h, src3, dst3, z2d, with_deg):
    f32 = jnp.float32
    mesh = plsc.VectorSubcoreMesh(core_axis_name="c", subcore_axis_name="s")
    idx_bufs = [pltpu.VMEM((CHUNK,), jnp.int32) for _ in range(8)]
    row_bufs = [pltpu.VMEM((CHUNK, D), f32) for _ in range(2)]
    sems = [pltpu.SemaphoreType.DMA for _ in range(8)]
    if with_deg:
        out_type = (jax.ShapeDtypeStruct((NC, N, D), f32),
                    jax.ShapeDtypeStruct((NC, N_PAD), f32))
        scratch = [
            pltpu.VMEM_SHARED((N_PAD, D), f32),
            pltpu.VMEM_SHARED((N_PAD,), f32),
            *idx_bufs, *row_bufs,
            pltpu.VMEM((CHUNK,), f32),
            pltpu.VMEM((ROW_CHUNK,), f32),
            *sems,
        ]
        fn = pl.kernel(functools.partial(_sc_agg_body, True),
                       out_type=out_type, mesh=mesh, scratch_types=scratch)
        return fn(h, src3, dst3, z2d)
    out_type = (jax.ShapeDtypeStruct((NC, N, D), f32),)
    scratch = [
        pltpu.VMEM_SHARED((N_PAD, D), f32),
        *idx_bufs, *row_bufs,
        *sems,
    ]
    fn = pl.kernel(functools.partial(_sc_agg_body, False),
                   out_type=out_type, mesh=mesh, scratch_types=scratch)
    return fn(h, src3, dst3, z2d)[0]


ROWS_TC = 1000


def _tc_pre_body(h_ref, wl_ref, b_ref, out_ref):
    out_ref[...] = jnp.dot(h_ref[...], wl_ref[...],
                           preferred_element_type=jnp.float32) + b_ref[...]


def _tc_post0_body(hw_ref, aggA_ref, aggB_ref, degA_ref, degB_ref,
                   wr_ref, out_ref, inv_ref):
    deg = jnp.maximum(degA_ref[0] + degB_ref[0], 1.0)
    inv = 1.0 / deg
    mean = (aggA_ref[0] + aggB_ref[0]) * inv
    acc = hw_ref[...] + jnp.dot(mean, wr_ref[...],
                                preferred_element_type=jnp.float32)
    out_ref[...] = jnp.maximum(acc, 0.0)
    inv_ref[...] = inv


def _tc_post_body(relu, hw_ref, aggA_ref, aggB_ref, inv_ref,
                  wr_ref, out_ref):
    mean = (aggA_ref[0] + aggB_ref[0]) * inv_ref[...]
    acc = hw_ref[...] + jnp.dot(mean, wr_ref[...],
                                preferred_element_type=jnp.float32)
    if relu:
        acc = jnp.maximum(acc, 0.0)
    out_ref[...] = acc


def _tc_common_specs():
    h_spec = pl.BlockSpec((ROWS_TC, D), lambda i: (i, 0))
    aggA = pl.BlockSpec((1, ROWS_TC, D), lambda i: (0, i, 0))
    aggB = pl.BlockSpec((1, ROWS_TC, D), lambda i: (1, i, 0))
    w_spec = pl.BlockSpec((D, D), lambda i: (0, 0))
    b_spec = pl.BlockSpec((1, D), lambda i: (0, 0))
    return h_spec, aggA, aggB, w_spec, b_spec


def _tc_pre(h, wl, b):
    h_spec, _, _, w_spec, b_spec = _tc_common_specs()
    return pl.pallas_call(
        _tc_pre_body,
        grid=(N // ROWS_TC,),
        in_specs=[h_spec, w_spec, b_spec],
        out_specs=h_spec,
        out_shape=jax.ShapeDtypeStruct((N, D), jnp.float32),
    )(h, wl, b.reshape(1, D))


def _tc_post0(hw, agg_p, deg_p, wr):
    f32 = jnp.float32
    h_spec, aggA, aggB, w_spec, _ = _tc_common_specs()
    degA = pl.BlockSpec((1, ROWS_TC, 1), lambda i: (0, i, 0))
    degB = pl.BlockSpec((1, ROWS_TC, 1), lambda i: (1, i, 0))
    inv_spec = pl.BlockSpec((ROWS_TC, 1), lambda i: (i, 0))
    return pl.pallas_call(
        _tc_post0_body,
        grid=(N // ROWS_TC,),
        in_specs=[h_spec, aggA, aggB, degA, degB, w_spec],
        out_specs=[h_spec, inv_spec],
        out_shape=[jax.ShapeDtypeStruct((N, D), f32),
                   jax.ShapeDtypeStruct((N, 1), f32)],
    )(hw, agg_p, agg_p, deg_p.reshape(NC, N, 1), deg_p.reshape(NC, N, 1),
      wr)


def _tc_post(hw, agg_p, inv_deg, wr, relu):
    h_spec, aggA, aggB, w_spec, _ = _tc_common_specs()
    inv_spec = pl.BlockSpec((ROWS_TC, 1), lambda i: (i, 0))
    return pl.pallas_call(
        functools.partial(_tc_post_body, relu),
        grid=(N // ROWS_TC,),
        in_specs=[h_spec, aggA, aggB, inv_spec, w_spec],
        out_specs=h_spec,
        out_shape=jax.ShapeDtypeStruct((N, D), jnp.float32),
    )(hw, agg_p, agg_p, inv_deg, wr)


def kernel(x, edge_index, Wl0, Wr0, b0, Wl1, Wr1, b1, Wl2, Wr2, b2):
    i32 = jnp.int32
    src = edge_index[0].reshape(NW, EDGES_PER_TILE)
    dst = edge_index[1].reshape(NW, EDGES_PER_TILE)
    k = jnp.arange(PAD_SLOTS, dtype=i32)[None, :]
    wv = jnp.arange(NW, dtype=i32)[:, None]
    pad_src = (k * 41 + wv * 13) % N
    pad_dst = N + (k + wv * 7) % (N_PAD - N)
    src3 = jnp.concatenate([src, pad_src], axis=1).reshape(NW, STEPS, CHUNK)
    dst3 = jnp.concatenate([dst, pad_dst], axis=1).reshape(NW, STEPS, CHUNK)
    z2d = jnp.zeros((N_PAD, D), jnp.float32)

    agg0, deg_pad = _sc_aggregate(x, src3, dst3, z2d, with_deg=True)
    hw0 = _tc_pre(x, Wl0, b0)
    deg_p = deg_pad[:, :N]
    h1, inv_deg = _tc_post0(hw0, agg0, deg_p, Wr0)

    agg1 = _sc_aggregate(h1, src3, dst3, z2d, with_deg=False)
    hw1 = _tc_pre(h1, Wl1, b1)
    h2 = _tc_post(hw1, agg1, inv_deg, Wr1, relu=True)

    agg2 = _sc_aggregate(h2, src3, dst3, z2d, with_deg=False)
    hw2 = _tc_pre(h2, Wl2, b2)
    return _tc_post(hw2, agg2, inv_deg, Wr2, relu=False)


# speedup vs baseline: 1.0325x; 1.0325x over previous
import functools

import jax
import jax.numpy as jnp
from jax import lax
from jax.experimental import pallas as pl
from jax.experimental.pallas import tpu as pltpu
from jax.experimental.pallas import tpu_sc as plsc

N = 10000
D = 128
E = 320000

NC = 2
NS = 16
NW = NC * NS
CHUNK = 80
EDGES_PER_TILE = E // NW
STEPS = 128
E_TILE_PAD = STEPS * CHUNK
PAD_SLOTS = E_TILE_PAD - EDGES_PER_TILE

ROW_CHUNK = 640
ROW_LAST = N - (NS - 1) * ROW_CHUNK
N_PAD = NS * ROW_CHUNK


def _sc_agg_body(with_deg, h_hbm, src_hbm, dst_hbm, z2d_hbm, *rest):
    if with_deg:
        agg_out, deg_out, acc_sh, deg_sh = rest[:4]
        r = rest[4:]
    else:
        agg_out, acc_sh = rest[:2]
        r = rest[2:]
    srcb = r[0:4]
    dstb = r[4:12]
    rows = r[12:16]
    k = 16
    if with_deg:
        ones_v, zbuf_v = r[16:18]
        k = 18
    srcsems = r[k:k + 4]
    dstsems = r[k + 4:k + 12]
    gsems = r[k + 12:k + 16]
    ssems = r[k + 16:k + 20]
    c = lax.axis_index("c")
    s = lax.axis_index("s")
    w = c * NS + s
    ebase = w * E_TILE_PAD

    def rows_split(fn):
        @pl.when(s < NS - 1)
        def _():
            fn(s * ROW_CHUNK, ROW_CHUNK)

        @pl.when(s == NS - 1)
        def _():
            fn((NS - 1) * ROW_CHUNK, ROW_LAST)

    def src_start(t, sj):
        pltpu.async_copy(src_hbm.at[pl.ds(ebase + t * CHUNK, CHUNK)],
                         srcb[sj], srcsems[sj])

    def src_wait(t, sj):
        pltpu.make_async_copy(src_hbm.at[pl.ds(ebase + t * CHUNK, CHUNK)],
                              srcb[sj], srcsems[sj]).wait()

    def dst_start(t, dj):
        pltpu.async_copy(dst_hbm.at[pl.ds(ebase + t * CHUNK, CHUNK)],
                         dstb[dj], dstsems[dj])

    def dst_wait(t, dj):
        pltpu.make_async_copy(dst_hbm.at[pl.ds(ebase + t * CHUNK, CHUNK)],
                              dstb[dj], dstsems[dj]).wait()

    def gather_start(sj):
        pltpu.async_copy(h_hbm.at[srcb[sj]], rows[sj], gsems[sj])

    def gather_wait(sj):
        pltpu.make_async_copy(h_hbm.at[srcb[sj]], rows[sj],
                              gsems[sj]).wait()

    def scatter_start(dj, rj):
        pltpu.async_copy(rows[rj], acc_sh.at[dstb[dj]], ssems[rj], add=True)
        if with_deg:
            pltpu.async_copy(ones_v, deg_sh.at[dstb[dj]], ssems[rj],
                             add=True)

    def scatter_wait(dj, rj):
        pltpu.make_async_copy(rows[rj], acc_sh.at[dstb[dj]],
                              ssems[rj]).wait()
        if with_deg:
            pltpu.make_async_copy(ones_v, deg_sh.at[dstb[dj]],
                                  ssems[rj]).wait()

    pltpu.sync_copy(z2d_hbm.at[pl.ds(s * ROW_CHUNK, ROW_CHUNK), :],
                    acc_sh.at[pl.ds(s * ROW_CHUNK, ROW_CHUNK), :])
    if with_deg:
        for k2 in range(ROW_CHUNK // 16):
            zbuf_v[pl.ds(k2 * 16, 16)] = jnp.zeros((16,), jnp.float32)
        pltpu.sync_copy(zbuf_v, deg_sh.at[pl.ds(s * ROW_CHUNK, ROW_CHUNK)])

        for k2 in range(CHUNK // 16):
            ones_v[pl.ds(k2 * 16, 16)] = jnp.ones((16,), jnp.float32)

    plsc.subcore_barrier()

    src_start(0, 0)
    src_start(1, 1)
    for d in range(4):
        dst_start(d, d)

    def group(g, carry):
        for j in range(8):
            t = g * 8 + j
            j4 = j % 4
            src_wait(t, j4)
            dst_wait(t, j)

            @pl.when(t >= 4)
            def _():
                scatter_wait((j - 4) % 8, j4)

            gather_start(j4)

            @pl.when(t >= 2)
            def _():
                gather_wait((j - 2) % 4)
                scatter_start((j - 2) % 8, (j - 2) % 4)

            @pl.when(t + 2 < STEPS)
            def _():
                src_start(t + 2, (j + 2) % 4)

            @pl.when(t + 4 < STEPS)
            def _():
                dst_start(t + 4, (j + 4) % 8)
        return carry

    lax.fori_loop(0, STEPS // 8, group, 0)
    gather_wait(2)
    scatter_start(6, 2)
    gather_wait(3)
    scatter_start(7, 3)
    scatter_wait(4, 0)
    scatter_wait(5, 1)
    scatter_wait(6, 2)
    scatter_wait(7, 3)

    plsc.subcore_barrier()

    rows_split(lambda b, n: pltpu.sync_copy(
        acc_sh.at[pl.ds(b, n), :], agg_out.at[c, pl.ds(b, n), :]))
    if with_deg:
        b = s * ROW_CHUNK
        pltpu.sync_copy(deg_sh.at[pl.ds(b, ROW_CHUNK)], zbuf_v)
        pltpu.sync_copy(zbuf_v, deg_out.at[c, pl.ds(b, ROW_CHUNK)])


def _sc_aggregate(h, src3, dst3, z2d, with_deg):
    f32 = jnp.float32
    mesh = plsc.VectorSubcoreMesh(core_axis_name="c", subcore_axis_name="s")
    idx_bufs = [pltpu.VMEM((CHUNK,), jnp.int32) for _ in range(12)]
    row_bufs = [pltpu.VMEM((CHUNK, D), f32) for _ in range(4)]
    sems = [pltpu.SemaphoreType.DMA for _ in range(20)]
    if with_deg:
        out_type = (jax.ShapeDtypeStruct((NC, N, D), f32),
                    jax.ShapeDtypeStruct((NC, N_PAD), f32))
        scratch = [
            pltpu.VMEM_SHARED((N_PAD, D), f32),
            pltpu.VMEM_SHARED((N_PAD,), f32),
            *idx_bufs, *row_bufs,
            pltpu.VMEM((CHUNK,), f32),
            pltpu.VMEM((ROW_CHUNK,), f32),
            *sems,
        ]
        fn = pl.kernel(functools.partial(_sc_agg_body, True),
                       out_type=out_type, mesh=mesh, scratch_types=scratch)
        return fn(h, src3, dst3, z2d)
    out_type = (jax.ShapeDtypeStruct((NC, N, D), f32),)
    scratch = [
        pltpu.VMEM_SHARED((N_PAD, D), f32),
        *idx_bufs, *row_bufs,
        *sems,
    ]
    fn = pl.kernel(functools.partial(_sc_agg_body, False),
                   out_type=out_type, mesh=mesh, scratch_types=scratch)
    return fn(h, src3, dst3, z2d)[0]


ROWS_TC = 1000


def _tc_pre_body(h_ref, wl_ref, b_ref, out_ref):
    out_ref[...] = jnp.dot(h_ref[...], wl_ref[...],
                           preferred_element_type=jnp.float32) + b_ref[...]


def _tc_post0_body(hw_ref, aggA_ref, aggB_ref, degA_ref, degB_ref,
                   wr_ref, out_ref, inv_ref):
    deg = jnp.maximum(degA_ref[0] + degB_ref[0], 1.0)
    inv = 1.0 / deg
    mean = (aggA_ref[0] + aggB_ref[0]) * inv
    acc = hw_ref[...] + jnp.dot(mean, wr_ref[...],
                                preferred_element_type=jnp.float32)
    out_ref[...] = jnp.maximum(acc, 0.0)
    inv_ref[...] = inv


def _tc_post_body(relu, hw_ref, aggA_ref, aggB_ref, inv_ref,
                  wr_ref, out_ref):
    mean = (aggA_ref[0] + aggB_ref[0]) * inv_ref[...]
    acc = hw_ref[...] + jnp.dot(mean, wr_ref[...],
                                preferred_element_type=jnp.float32)
    if relu:
        acc = jnp.maximum(acc, 0.0)
    out_ref[...] = acc


def _tc_common_specs():
    h_spec = pl.BlockSpec((ROWS_TC, D), lambda i: (i, 0))
    aggA = pl.BlockSpec((1, ROWS_TC, D), lambda i: (0, i, 0))
    aggB = pl.BlockSpec((1, ROWS_TC, D), lambda i: (1, i, 0))
    w_spec = pl.BlockSpec((D, D), lambda i: (0, 0))
    b_spec = pl.BlockSpec((1, D), lambda i: (0, 0))
    return h_spec, aggA, aggB, w_spec, b_spec


def _tc_pre(h, wl, b):
    h_spec, _, _, w_spec, b_spec = _tc_common_specs()
    return pl.pallas_call(
        _tc_pre_body,
        grid=(N // ROWS_TC,),
        in_specs=[h_spec, w_spec, b_spec],
        out_specs=h_spec,
        out_shape=jax.ShapeDtypeStruct((N, D), jnp.float32),
    )(h, wl, b.reshape(1, D))


def _tc_post0(hw, agg_p, deg_p, wr):
    f32 = jnp.float32
    h_spec, aggA, aggB, w_spec, _ = _tc_common_specs()
    degA = pl.BlockSpec((1, ROWS_TC, 1), lambda i: (0, i, 0))
    degB = pl.BlockSpec((1, ROWS_TC, 1), lambda i: (1, i, 0))
    inv_spec = pl.BlockSpec((ROWS_TC, 1), lambda i: (i, 0))
    return pl.pallas_call(
        _tc_post0_body,
        grid=(N // ROWS_TC,),
        in_specs=[h_spec, aggA, aggB, degA, degB, w_spec],
        out_specs=[h_spec, inv_spec],
        out_shape=[jax.ShapeDtypeStruct((N, D), f32),
                   jax.ShapeDtypeStruct((N, 1), f32)],
    )(hw, agg_p, agg_p, deg_p.reshape(NC, N, 1), deg_p.reshape(NC, N, 1),
      wr)


def _tc_post(hw, agg_p, inv_deg, wr, relu):
    h_spec, aggA, aggB, w_spec, _ = _tc_common_specs()
    inv_spec = pl.BlockSpec((ROWS_TC, 1), lambda i: (i, 0))
    return pl.pallas_call(
        functools.partial(_tc_post_body, relu),
        grid=(N // ROWS_TC,),
        in_specs=[h_spec, aggA, aggB, inv_spec, w_spec],
        out_specs=h_spec,
        out_shape=jax.ShapeDtypeStruct((N, D), jnp.float32),
    )(hw, agg_p, agg_p, inv_deg, wr)


def kernel(x, edge_index, Wl0, Wr0, b0, Wl1, Wr1, b1, Wl2, Wr2, b2):
    i32 = jnp.int32
    src = edge_index[0].reshape(NW, EDGES_PER_TILE)
    dst = edge_index[1].reshape(NW, EDGES_PER_TILE)
    k = jnp.arange(PAD_SLOTS, dtype=i32)[None, :]
    wv = jnp.arange(NW, dtype=i32)[:, None]
    pad_src = (k * 41 + wv * 13) % N
    pad_dst = N + (k + wv * 7) % (N_PAD - N)
    src3 = jnp.concatenate([src, pad_src], axis=1).reshape(-1)
    dst3 = jnp.concatenate([dst, pad_dst], axis=1).reshape(-1)
    z2d = jnp.zeros((N_PAD, D), jnp.float32)

    agg0, deg_pad = _sc_aggregate(x, src3, dst3, z2d, with_deg=True)
    hw0 = _tc_pre(x, Wl0, b0)
    deg_p = deg_pad[:, :N]
    h1, inv_deg = _tc_post0(hw0, agg0, deg_p, Wr0)

    agg1 = _sc_aggregate(h1, src3, dst3, z2d, with_deg=False)
    hw1 = _tc_pre(h1, Wl1, b1)
    h2 = _tc_post(hw1, agg1, inv_deg, Wr1, relu=True)

    agg2 = _sc_aggregate(h2, src3, dst3, z2d, with_deg=False)
    hw2 = _tc_pre(h2, Wl2, b2)
    return _tc_post(hw2, agg2, inv_deg, Wr2, relu=False)


# speedup vs baseline: 1.1119x; 1.0769x over previous
import functools

import jax
import jax.numpy as jnp
from jax import lax
from jax.experimental import pallas as pl
from jax.experimental.pallas import tpu as pltpu
from jax.experimental.pallas import tpu_sc as plsc

N = 10000
D = 128
E = 320000

NC = 2
NS = 16
NW = NC * NS
CHUNK = 80
EDGES_PER_TILE = E // NW
STEPS = 128
E_TILE_PAD = STEPS * CHUNK
PAD_SLOTS = E_TILE_PAD - EDGES_PER_TILE

ROW_CHUNK = 640
ROW_LAST = N - (NS - 1) * ROW_CHUNK
N_PAD = NS * ROW_CHUNK


def _sc_agg_body(with_deg, h_hbm, src_hbm, dst_hbm, z2d_hbm, *rest):
    if with_deg:
        agg_out, deg_out, acc_sh, deg_sh = rest[:4]
        r = rest[4:]
    else:
        agg_out, acc_sh = rest[:2]
        r = rest[2:]
    srcb = r[0:4]
    dstb = r[4:12]
    rows = r[12:16]
    k = 16
    if with_deg:
        ones_v, zbuf_v = r[16:18]
        k = 18
    srcsems = r[k:k + 4]
    dstsems = r[k + 4:k + 12]
    gsems = r[k + 12:k + 16]
    ssems = r[k + 16:k + 20]
    c = lax.axis_index("c")
    s = lax.axis_index("s")
    w = c * NS + s
    ebase = w * E_TILE_PAD

    def rows_split(fn):
        @pl.when(s < NS - 1)
        def _():
            fn(s * ROW_CHUNK, ROW_CHUNK)

        @pl.when(s == NS - 1)
        def _():
            fn((NS - 1) * ROW_CHUNK, ROW_LAST)

    def src_start(t, sj):
        pltpu.async_copy(src_hbm.at[pl.ds(ebase + t * CHUNK, CHUNK)],
                         srcb[sj], srcsems[sj])

    def src_wait(t, sj):
        pltpu.make_async_copy(src_hbm.at[pl.ds(ebase + t * CHUNK, CHUNK)],
                              srcb[sj], srcsems[sj]).wait()

    def dst_start(t, dj):
        pltpu.async_copy(dst_hbm.at[pl.ds(ebase + t * CHUNK, CHUNK)],
                         dstb[dj], dstsems[dj])

    def dst_wait(t, dj):
        pltpu.make_async_copy(dst_hbm.at[pl.ds(ebase + t * CHUNK, CHUNK)],
                              dstb[dj], dstsems[dj]).wait()

    def gather_start(sj):
        pltpu.async_copy(h_hbm.at[srcb[sj]], rows[sj], gsems[sj])

    def gather_wait(sj):
        pltpu.make_async_copy(h_hbm.at[srcb[sj]], rows[sj],
                              gsems[sj]).wait()

    def scatter_start(dj, rj):
        pltpu.async_copy(rows[rj], acc_sh.at[dstb[dj]], ssems[rj], add=True)
        if with_deg:
            pltpu.async_copy(ones_v, deg_sh.at[dstb[dj]], ssems[rj],
                             add=True)

    def scatter_wait(dj, rj):
        pltpu.make_async_copy(rows[rj], acc_sh.at[dstb[dj]],
                              ssems[rj]).wait()
        if with_deg:
            pltpu.make_async_copy(ones_v, deg_sh.at[dstb[dj]],
                                  ssems[rj]).wait()

    pltpu.sync_copy(z2d_hbm.at[pl.ds(s * ROW_CHUNK, ROW_CHUNK), :],
                    acc_sh.at[pl.ds(s * ROW_CHUNK, ROW_CHUNK), :])
    if with_deg:
        for k2 in range(ROW_CHUNK // 16):
            zbuf_v[pl.ds(k2 * 16, 16)] = jnp.zeros((16,), jnp.float32)
        pltpu.sync_copy(zbuf_v, deg_sh.at[pl.ds(s * ROW_CHUNK, ROW_CHUNK)])

        for k2 in range(CHUNK // 16):
            ones_v[pl.ds(k2 * 16, 16)] = jnp.ones((16,), jnp.float32)

    plsc.subcore_barrier()

    src_start(0, 0)
    src_start(1, 1)
    for d in range(4):
        dst_start(d, d)

    def group(g, carry):
        for j in range(8):
            t = g * 8 + j
            j4 = j % 4
            src_wait(t, j4)
            dst_wait(t, j)

            @pl.when(t >= 4)
            def _():
                scatter_wait((j - 4) % 8, j4)

            gather_start(j4)

            @pl.when(t >= 3)
            def _():
                gather_wait((j - 3) % 4)
                scatter_start((j - 3) % 8, (j - 3) % 4)

            @pl.when(t + 2 < STEPS)
            def _():
                src_start(t + 2, (j + 2) % 4)

            @pl.when(t + 4 < STEPS)
            def _():
                dst_start(t + 4, (j + 4) % 8)
        return carry

    lax.fori_loop(0, STEPS // 8, group, 0)
    gather_wait(1)
    scatter_start(5, 1)
    gather_wait(2)
    scatter_start(6, 2)
    gather_wait(3)
    scatter_start(7, 3)
    scatter_wait(4, 0)
    scatter_wait(5, 1)
    scatter_wait(6, 2)
    scatter_wait(7, 3)

    plsc.subcore_barrier()

    rows_split(lambda b, n: pltpu.sync_copy(
        acc_sh.at[pl.ds(b, n), :], agg_out.at[c, pl.ds(b, n), :]))
    if with_deg:
        b = s * ROW_CHUNK
        pltpu.sync_copy(deg_sh.at[pl.ds(b, ROW_CHUNK)], zbuf_v)
        pltpu.sync_copy(zbuf_v, deg_out.at[c, pl.ds(b, ROW_CHUNK)])


def _sc_aggregate(h, src3, dst3, z2d, with_deg):
    f32 = jnp.float32
    mesh = plsc.VectorSubcoreMesh(core_axis_name="c", subcore_axis_name="s")
    idx_bufs = [pltpu.VMEM((CHUNK,), jnp.int32) for _ in range(12)]
    row_bufs = [pltpu.VMEM((CHUNK, D), f32) for _ in range(4)]
    sems = [pltpu.SemaphoreType.DMA for _ in range(20)]
    if with_deg:
        out_type = (jax.ShapeDtypeStruct((NC, N, D), f32),
                    jax.ShapeDtypeStruct((NC, N_PAD), f32))
        scratch = [
            pltpu.VMEM_SHARED((N_PAD, D), f32),
            pltpu.VMEM_SHARED((N_PAD,), f32),
            *idx_bufs, *row_bufs,
            pltpu.VMEM((CHUNK,), f32),
            pltpu.VMEM((ROW_CHUNK,), f32),
            *sems,
        ]
        fn = pl.kernel(functools.partial(_sc_agg_body, True),
                       out_type=out_type, mesh=mesh, scratch_types=scratch)
        return fn(h, src3, dst3, z2d)
    out_type = (jax.ShapeDtypeStruct((NC, N, D), f32),)
    scratch = [
        pltpu.VMEM_SHARED((N_PAD, D), f32),
        *idx_bufs, *row_bufs,
        *sems,
    ]
    fn = pl.kernel(functools.partial(_sc_agg_body, False),
                   out_type=out_type, mesh=mesh, scratch_types=scratch)
    return fn(h, src3, dst3, z2d)[0]


ROWS_TC = 1000


def _tc_pre_body(h_ref, wl_ref, b_ref, out_ref):
    out_ref[...] = jnp.dot(h_ref[...], wl_ref[...],
                           preferred_element_type=jnp.float32) + b_ref[...]


def _tc_post0_body(hw_ref, aggA_ref, aggB_ref, degA_ref, degB_ref,
                   wr_ref, out_ref, inv_ref):
    deg = jnp.maximum(degA_ref[0] + degB_ref[0], 1.0)
    inv = 1.0 / deg
    mean = (aggA_ref[0] + aggB_ref[0]) * inv
    acc = hw_ref[...] + jnp.dot(mean, wr_ref[...],
                                preferred_element_type=jnp.float32)
    out_ref[...] = jnp.maximum(acc, 0.0)
    inv_ref[...] = inv


def _tc_post_body(relu, hw_ref, aggA_ref, aggB_ref, inv_ref,
                  wr_ref, out_ref):
    mean = (aggA_ref[0] + aggB_ref[0]) * inv_ref[...]
    acc = hw_ref[...] + jnp.dot(mean, wr_ref[...],
                                preferred_element_type=jnp.float32)
    if relu:
        acc = jnp.maximum(acc, 0.0)
    out_ref[...] = acc


def _tc_common_specs():
    h_spec = pl.BlockSpec((ROWS_TC, D), lambda i: (i, 0))
    aggA = pl.BlockSpec((1, ROWS_TC, D), lambda i: (0, i, 0))
    aggB = pl.BlockSpec((1, ROWS_TC, D), lambda i: (1, i, 0))
    w_spec = pl.BlockSpec((D, D), lambda i: (0, 0))
    b_spec = pl.BlockSpec((1, D), lambda i: (0, 0))
    return h_spec, aggA, aggB, w_spec, b_spec


def _tc_pre(h, wl, b):
    h_spec, _, _, w_spec, b_spec = _tc_common_specs()
    return pl.pallas_call(
        _tc_pre_body,
        grid=(N // ROWS_TC,),
        in_specs=[h_spec, w_spec, b_spec],
        out_specs=h_spec,
        out_shape=jax.ShapeDtypeStruct((N, D), jnp.float32),
    )(h, wl, b.reshape(1, D))


def _tc_post0(hw, agg_p, deg_p, wr):
    f32 = jnp.float32
    h_spec, aggA, aggB, w_spec, _ = _tc_common_specs()
    degA = pl.BlockSpec((1, ROWS_TC, 1), lambda i: (0, i, 0))
    degB = pl.BlockSpec((1, ROWS_TC, 1), lambda i: (1, i, 0))
    inv_spec = pl.BlockSpec((ROWS_TC, 1), lambda i: (i, 0))
    return pl.pallas_call(
        _tc_post0_body,
        grid=(N // ROWS_TC,),
        in_specs=[h_spec, aggA, aggB, degA, degB, w_spec],
        out_specs=[h_spec, inv_spec],
        out_shape=[jax.ShapeDtypeStruct((N, D), f32),
                   jax.ShapeDtypeStruct((N, 1), f32)],
    )(hw, agg_p, agg_p, deg_p.reshape(NC, N, 1), deg_p.reshape(NC, N, 1),
      wr)


def _tc_post(hw, agg_p, inv_deg, wr, relu):
    h_spec, aggA, aggB, w_spec, _ = _tc_common_specs()
    inv_spec = pl.BlockSpec((ROWS_TC, 1), lambda i: (i, 0))
    return pl.pallas_call(
        functools.partial(_tc_post_body, relu),
        grid=(N // ROWS_TC,),
        in_specs=[h_spec, aggA, aggB, inv_spec, w_spec],
        out_specs=h_spec,
        out_shape=jax.ShapeDtypeStruct((N, D), jnp.float32),
    )(hw, agg_p, agg_p, inv_deg, wr)


def kernel(x, edge_index, Wl0, Wr0, b0, Wl1, Wr1, b1, Wl2, Wr2, b2):
    i32 = jnp.int32
    src = edge_index[0].reshape(NW, EDGES_PER_TILE)
    dst = edge_index[1].reshape(NW, EDGES_PER_TILE)
    k = jnp.arange(PAD_SLOTS, dtype=i32)[None, :]
    wv = jnp.arange(NW, dtype=i32)[:, None]
    pad_src = (k * 41 + wv * 13) % N
    pad_dst = N + (k + wv * 7) % (N_PAD - N)
    src3 = jnp.concatenate([src, pad_src], axis=1).reshape(-1)
    dst3 = jnp.concatenate([dst, pad_dst], axis=1).reshape(-1)
    z2d = jnp.zeros((N_PAD, D), jnp.float32)

    agg0, deg_pad = _sc_aggregate(x, src3, dst3, z2d, with_deg=True)
    hw0 = _tc_pre(x, Wl0, b0)
    deg_p = deg_pad[:, :N]
    h1, inv_deg = _tc_post0(hw0, agg0, deg_p, Wr0)

    agg1 = _sc_aggregate(h1, src3, dst3, z2d, with_deg=False)
    hw1 = _tc_pre(h1, Wl1, b1)
    h2 = _tc_post(hw1, agg1, inv_deg, Wr1, relu=True)

    agg2 = _sc_aggregate(h2, src3, dst3, z2d, with_deg=False)
    hw2 = _tc_pre(h2, Wl2, b2)
    return _tc_post(hw2, agg2, inv_deg, Wr2, relu=False)
